# baseline (speedup 1.0000x reference)
import functools

import jax
import jax.numpy as jnp
from jax import lax
from jax.experimental import pallas as pl
from jax.experimental.pallas import tpu as pltpu
from jax.experimental.pallas import tpu_sc as plsc

N = 10000
D = 128
NC = 2
NS = 16
NW = NC * NS
CH = 128
NP = 10240
DUMMY_DST = N + 64
DEGW = 16


def _pad_edges(src, dst, e_pad):
    ar = jnp.arange(e_pad, dtype=jnp.int32)
    src_p = jnp.concatenate([src, ar % N])
    dst_p = jnp.concatenate([dst, N + (ar % (NP - N))])
    return src_p, dst_p


NB = 2
NQ = 2
STRIPE = NP // NS


def _sc_scatter_body(nch, table, src_hbm, dst_hbm, zeros_hbm, pout,
                     src_v, dst_v, r0_, r1_, acc_sh, g0, g1, s0, s1):
    c = lax.axis_index("c")
    s = lax.axis_index("s")
    w = c * NS + s
    rows = [r0_, r1_]
    gsem = [g0, g1]
    ssem = [s0, s1]
    per_q = nch // NQ

    base_r = s * STRIPE
    pltpu.sync_copy(zeros_hbm, acc_sh.at[pl.ds(base_r, STRIPE)])
    plsc.subcore_barrier()

    def _quarter(qi, _):
        roff = w * nch + qi * per_q
        hs = pltpu.async_copy(src_hbm.at[pl.ds(roff, per_q)], src_v, g0)
        hd = pltpu.async_copy(dst_hbm.at[pl.ds(roff, per_q)], dst_v, g1)
        hs.wait()
        hd.wait()

        def _group(gi, _):
            hg = []
            for b in range(NB):
                j = gi * NB + b
                hg.append(pltpu.async_copy(table.at[src_v.at[j]], rows[b],
                                           gsem[b]))
            hsc = []
            for b in range(NB):
                j = gi * NB + b
                hg[b].wait()
                hsc.append(pltpu.async_copy(rows[b], acc_sh.at[dst_v.at[j]],
                                            ssem[b], add=True))
            for b in range(NB):
                hsc[b].wait()
            return 0
        return lax.fori_loop(0, per_q // NB, _group, 0)
    lax.fori_loop(0, NQ, _quarter, 0)
    plsc.subcore_barrier()

    pltpu.sync_copy(acc_sh.at[pl.ds(base_r, STRIPE)],
                    pout.at[c, pl.ds(base_r, STRIPE)])


def _make_sc_scatter(nch):
    mesh = plsc.VectorSubcoreMesh(core_axis_name="c", subcore_axis_name="s")
    return pl.kernel(
        functools.partial(_sc_scatter_body, nch),
        out_type=[jax.ShapeDtypeStruct((NC, NP, D), jnp.float32)],
        mesh=mesh,
        scratch_types=[pltpu.VMEM((nch // NQ, CH), jnp.int32),
                       pltpu.VMEM((nch // NQ, CH), jnp.int32)]
                      + [pltpu.VMEM((CH, D), jnp.float32)] * NB
                      + [pltpu.VMEM_SHARED((NP, D), jnp.float32)]
                      + [pltpu.SemaphoreType.DMA] * (2 * NB))


def _sc_deg_body(nch, dst_hbm, zeros_hbm, ones_hbm, dout,
                 dst_v, ones_v, acc_sh, s0, s1):
    c = lax.axis_index("c")
    s = lax.axis_index("s")
    w = c * NS + s
    ssem = [s0, s1]
    per_q = nch // NQ

    base_r = s * STRIPE
    pltpu.sync_copy(zeros_hbm, acc_sh.at[pl.ds(base_r, STRIPE)])
    pltpu.sync_copy(ones_hbm, ones_v)
    plsc.subcore_barrier()

    def _quarter(qi, _):
        roff = w * nch + qi * per_q
        pltpu.sync_copy(dst_hbm.at[pl.ds(roff, per_q)], dst_v)

        def _group(gi, _):
            hsc = []
            for b in range(NB):
                j = gi * NB + b
                hsc.append(pltpu.async_copy(ones_v, acc_sh.at[dst_v.at[j]],
                                            ssem[b], add=True))
            for b in range(NB):
                hsc[b].wait()
            return 0
        return lax.fori_loop(0, per_q // NB, _group, 0)
    lax.fori_loop(0, NQ, _quarter, 0)
    plsc.subcore_barrier()

    pltpu.sync_copy(acc_sh.at[pl.ds(base_r, STRIPE)],
                    dout.at[c, pl.ds(base_r, STRIPE)])


def _make_sc_deg(nch):
    mesh = plsc.VectorSubcoreMesh(core_axis_name="c", subcore_axis_name="s")
    return pl.kernel(
        functools.partial(_sc_deg_body, nch),
        out_type=[jax.ShapeDtypeStruct((NC, NP, D), jnp.float32)],
        mesh=mesh,
        scratch_types=[pltpu.VMEM((nch // NQ, CH), jnp.int32),
                       pltpu.VMEM((CH, D), jnp.float32),
                       pltpu.VMEM_SHARED((NP, D), jnp.float32)]
                      + [pltpu.SemaphoreType.DMA] * NB)


BR = 400


def _tc1_body(x, xorg, wres, win, bin_, wg0, xr, u1):
    xr[...] = jnp.dot(xorg[...], wres[...], preferred_element_type=jnp.float32)
    h0 = jnp.maximum(
        jnp.dot(x[...], win[...], preferred_element_type=jnp.float32)
        + bin_[...], 0.0)
    u1[...] = jnp.dot(h0, wg0[...], preferred_element_type=jnp.float32)


def _tc2_body(rp0, rp1, dg0, dg1, u1, res, t1, dinv):
    res[...] = rp0[0] + rp1[0]
    deg = dg0[0][:, 0:1] + dg1[0][:, 0:1] + 1.0
    dv = lax.rsqrt(deg)
    dinv[...] = jnp.broadcast_to(dv, (BR, D))
    t1[...] = dv * u1[...]


def _layer_math(sp0, sp1, t, dinv, b, g, be):
    a = dinv[...] * (sp0[0] + sp1[0] + t[...]) + b[...]
    mu = jnp.mean(a, axis=-1, keepdims=True)
    var = jnp.mean((a - mu) ** 2, axis=-1, keepdims=True)
    xhat = (a - mu) * lax.rsqrt(var + 1e-5) * g[...] + be[...]
    return jnp.maximum(xhat, 0.0)


def _tc_mid_body(sp0, sp1, t, dinv, b, g, be, wn, tn):
    h = _layer_math(sp0, sp1, t, dinv, b, g, be)
    tn[...] = dinv[...] * jnp.dot(h, wn[...],
                                  preferred_element_type=jnp.float32)


def _tc_fin_body(sp0, sp1, t, dinv, b, g, be, wlin, blin, out):
    h = _layer_math(sp0, sp1, t, dinv, b, g, be)
    out[...] = (jnp.dot(h, wlin[...], preferred_element_type=jnp.float32)
                + blin[...])


def _row_spec():
    return pl.BlockSpec((BR, D), lambda i: (i, 0))


def _part_spec(core):
    return pl.BlockSpec((1, BR, D), lambda i, c=core: (c, i, 0))


def _full_spec(shape):
    return pl.BlockSpec(shape, lambda i: tuple(0 for _ in shape))


def _vec_spec():
    return pl.BlockSpec((1, D), lambda i: (0, 0))


def kernel(x, x_org, edge_index, W_in, b_in, W_g0, b_g0, ln_g0, ln_b0,
           W_g1, b_g1, ln_g1, ln_b1, W_g2, b_g2, ln_g2, ln_b2,
           W_lin, b_lin, W_res):
    n = x.shape[0]
    e = edge_index.shape[1]
    src, dst = edge_index[0], edge_index[1]
    nch0 = -(-e // (NW * CH))
    nch = -(-nch0 // NB) * NB
    ep = nch * CH * NW
    src_p, dst_p = _pad_edges(src, dst, ep - e)
    src_p = src_p.reshape(NW * nch, CH)
    dst_p = dst_p.reshape(NW * nch, CH)

    grid = n // BR
    b_in2 = b_in.reshape(1, D)
    b_g02 = b_g0.reshape(1, D)
    b_g12 = b_g1.reshape(1, D)
    b_g22 = b_g2.reshape(1, D)
    g0 = ln_g0.reshape(1, D)
    be0 = ln_b0.reshape(1, D)
    g1 = ln_g1.reshape(1, D)
    be1 = ln_b1.reshape(1, D)
    g2 = ln_g2.reshape(1, D)
    be2 = ln_b2.reshape(1, D)
    b_lin2 = b_lin.reshape(1, D)

    zeros_blk = jnp.zeros((STRIPE, D), jnp.float32)
    ones_blk = jnp.ones((CH, D), jnp.float32)

    dg, = _make_sc_deg(nch)(dst_p, zeros_blk, ones_blk)

    xr, u1 = pl.pallas_call(
        _tc1_body,
        grid=(grid,),
        in_specs=[_row_spec(), _row_spec(), _full_spec((D, D)),
                  _full_spec((D, D)), _vec_spec(), _full_spec((D, D))],
        out_specs=[_row_spec(), _row_spec()],
        out_shape=[jax.ShapeDtypeStruct((n, D), jnp.float32),
                   jax.ShapeDtypeStruct((n, D), jnp.float32)],
    )(x, x_org, W_res, W_in, b_in2, W_g0)

    sc_gs = _make_sc_scatter(nch)

    rp, = sc_gs(xr, src_p, dst_p, zeros_blk)

    res, t1, dinv = pl.pallas_call(
        _tc2_body,
        grid=(grid,),
        in_specs=[_part_spec(0), _part_spec(1),
                  _part_spec(0), _part_spec(1), _row_spec()],
        out_specs=[_row_spec(), _row_spec(), _row_spec()],
        out_shape=[jax.ShapeDtypeStruct((n, D), jnp.float32),
                   jax.ShapeDtypeStruct((n, D), jnp.float32),
                   jax.ShapeDtypeStruct((n, D), jnp.float32)],
    )(rp, rp, dg, dg, u1)

    def mid_layer(t, b2, g, be, wn):
        sp, = sc_gs(t, src_p, dst_p, zeros_blk)
        return pl.pallas_call(
            _tc_mid_body,
            grid=(grid,),
            in_specs=[_part_spec(0), _part_spec(1), _row_spec(), _row_spec(),
                      _vec_spec(), _vec_spec(), _vec_spec(),
                      _full_spec((D, D))],
            out_specs=[_row_spec()],
            out_shape=[jax.ShapeDtypeStruct((n, D), jnp.float32)],
        )(sp, sp, t, dinv, b2, g, be, wn)[0]

    t2 = mid_layer(t1, b_g02, g0, be0, W_g1)
    t3 = mid_layer(t2, b_g12, g1, be1, W_g2)

    sp, = sc_gs(t3, src_p, dst_p, zeros_blk)
    out = pl.pallas_call(
        _tc_fin_body,
        grid=(grid,),
        in_specs=[_part_spec(0), _part_spec(1), _row_spec(), _row_spec(),
                  _vec_spec(), _vec_spec(), _vec_spec(),
                  _full_spec((D, D)), _vec_spec()],
        out_specs=[_row_spec()],
        out_shape=[jax.ShapeDtypeStruct((n, D), jnp.float32)],
    )(sp, sp, t3, dinv, b_g22, g2, be2, W_lin, b_lin2)[0]

    return (out, res)


# speedup vs baseline: 1.0097x; 1.0097x over previous
import functools

import jax
import jax.numpy as jnp
from jax import lax
from jax.experimental import pallas as pl
from jax.experimental.pallas import tpu as pltpu
from jax.experimental.pallas import tpu_sc as plsc

N = 10000
D = 128
NC = 2
NS = 16
NW = NC * NS
CH = 128
NP = 10240
DUMMY_DST = N + 64
DEGW = 16


def _pad_edges(src, dst, e_pad):
    ar = jnp.arange(e_pad, dtype=jnp.int32)
    src_p = jnp.concatenate([src, ar % N])
    dst_p = jnp.concatenate([dst, N + (ar % (NP - N))])
    return src_p, dst_p


NB = 2
NQ = 2
STRIPE = NP // NS


def _sc_scatter_body(nch, table, src_hbm, dst_hbm, zeros_hbm, junk_hbm, pout,
                     src_v, dst_v, junk_v, r0_, r1_, acc_sh, g0, g1, s0, s1):
    c = lax.axis_index("c")
    s = lax.axis_index("s")
    w = c * NS + s
    rows = [r0_, r1_]
    gsem = [g0, g1]
    ssem = [s0, s1]
    per_q = nch // NQ

    base_r = s * STRIPE
    pltpu.sync_copy(junk_hbm, junk_v)
    pltpu.sync_copy(zeros_hbm, acc_sh.at[pl.ds(base_r, STRIPE)])
    plsc.subcore_barrier()

    for b in range(NB):
        pltpu.async_copy(rows[b], acc_sh.at[junk_v.at[b]], ssem[b], add=True)

    def _quarter(qi, _):
        roff = w * nch + qi * per_q
        hs = pltpu.async_copy(src_hbm.at[pl.ds(roff, per_q)], src_v, g0)
        hd = pltpu.async_copy(dst_hbm.at[pl.ds(roff, per_q)], dst_v, g1)
        hs.wait()
        hd.wait()

        def _group(gi, _):
            hg = []
            for b in range(NB):
                j = gi * NB + b
                pltpu.make_async_copy(rows[b], acc_sh.at[dst_v.at[j]],
                                      ssem[b]).wait()
                hg.append(pltpu.async_copy(table.at[src_v.at[j]], rows[b],
                                           gsem[b]))
            for b in range(NB):
                j = gi * NB + b
                hg[b].wait()
                pltpu.async_copy(rows[b], acc_sh.at[dst_v.at[j]],
                                 ssem[b], add=True)
            return 0
        return lax.fori_loop(0, per_q // NB, _group, 0)
    lax.fori_loop(0, NQ, _quarter, 0)

    for b in range(NB):
        pltpu.make_async_copy(rows[b], acc_sh.at[junk_v.at[b]],
                              ssem[b]).wait()
    plsc.subcore_barrier()

    pltpu.sync_copy(acc_sh.at[pl.ds(base_r, STRIPE)],
                    pout.at[c, pl.ds(base_r, STRIPE)])


def _make_sc_scatter(nch):
    mesh = plsc.VectorSubcoreMesh(core_axis_name="c", subcore_axis_name="s")
    return pl.kernel(
        functools.partial(_sc_scatter_body, nch),
        out_type=[jax.ShapeDtypeStruct((NC, NP, D), jnp.float32)],
        mesh=mesh,
        scratch_types=[pltpu.VMEM((nch // NQ, CH), jnp.int32),
                       pltpu.VMEM((nch // NQ, CH), jnp.int32),
                       pltpu.VMEM((NB, CH), jnp.int32)]
                      + [pltpu.VMEM((CH, D), jnp.float32)] * NB
                      + [pltpu.VMEM_SHARED((NP, D), jnp.float32)]
                      + [pltpu.SemaphoreType.DMA] * (2 * NB))


def _sc_deg_body(nch, dst_hbm, zeros_hbm, ones_hbm, dout,
                 dst_v, ones_v, acc_sh, s0, s1):
    c = lax.axis_index("c")
    s = lax.axis_index("s")
    w = c * NS + s
    ssem = [s0, s1]
    per_q = nch // NQ

    base_r = s * STRIPE
    pltpu.sync_copy(zeros_hbm, acc_sh.at[pl.ds(base_r, STRIPE)])
    pltpu.sync_copy(ones_hbm, ones_v)
    plsc.subcore_barrier()

    def _quarter(qi, _):
        roff = w * nch + qi * per_q
        pltpu.sync_copy(dst_hbm.at[pl.ds(roff, per_q)], dst_v)

        def _group(gi, _):
            hsc = []
            for b in range(NB):
                j = gi * NB + b
                hsc.append(pltpu.async_copy(ones_v, acc_sh.at[dst_v.at[j]],
                                            ssem[b], add=True))
            for b in range(NB):
                hsc[b].wait()
            return 0
        return lax.fori_loop(0, per_q // NB, _group, 0)
    lax.fori_loop(0, NQ, _quarter, 0)
    plsc.subcore_barrier()

    pltpu.sync_copy(acc_sh.at[pl.ds(base_r, STRIPE)],
                    dout.at[c, pl.ds(base_r, STRIPE)])


def _make_sc_deg(nch):
    mesh = plsc.VectorSubcoreMesh(core_axis_name="c", subcore_axis_name="s")
    return pl.kernel(
        functools.partial(_sc_deg_body, nch),
        out_type=[jax.ShapeDtypeStruct((NC, NP, D), jnp.float32)],
        mesh=mesh,
        scratch_types=[pltpu.VMEM((nch // NQ, CH), jnp.int32),
                       pltpu.VMEM((CH, D), jnp.float32),
                       pltpu.VMEM_SHARED((NP, D), jnp.float32)]
                      + [pltpu.SemaphoreType.DMA] * NB)


BR = 400


def _tc1_body(x, xorg, wres, win, bin_, wg0, xr, u1):
    xr[...] = jnp.dot(xorg[...], wres[...], preferred_element_type=jnp.float32)
    h0 = jnp.maximum(
        jnp.dot(x[...], win[...], preferred_element_type=jnp.float32)
        + bin_[...], 0.0)
    u1[...] = jnp.dot(h0, wg0[...], preferred_element_type=jnp.float32)


def _tc2_body(rp0, rp1, dg0, dg1, u1, res, t1, dinv):
    res[...] = rp0[0] + rp1[0]
    deg = dg0[0][:, 0:1] + dg1[0][:, 0:1] + 1.0
    dv = lax.rsqrt(deg)
    dinv[...] = jnp.broadcast_to(dv, (BR, D))
    t1[...] = dv * u1[...]


def _layer_math(sp0, sp1, t, dinv, b, g, be):
    a = dinv[...] * (sp0[0] + sp1[0] + t[...]) + b[...]
    mu = jnp.mean(a, axis=-1, keepdims=True)
    var = jnp.mean((a - mu) ** 2, axis=-1, keepdims=True)
    xhat = (a - mu) * lax.rsqrt(var + 1e-5) * g[...] + be[...]
    return jnp.maximum(xhat, 0.0)


def _tc_mid_body(sp0, sp1, t, dinv, b, g, be, wn, tn):
    h = _layer_math(sp0, sp1, t, dinv, b, g, be)
    tn[...] = dinv[...] * jnp.dot(h, wn[...],
                                  preferred_element_type=jnp.float32)


def _tc_fin_body(sp0, sp1, t, dinv, b, g, be, wlin, blin, out):
    h = _layer_math(sp0, sp1, t, dinv, b, g, be)
    out[...] = (jnp.dot(h, wlin[...], preferred_element_type=jnp.float32)
                + blin[...])


def _row_spec():
    return pl.BlockSpec((BR, D), lambda i: (i, 0))


def _part_spec(core):
    return pl.BlockSpec((1, BR, D), lambda i, c=core: (c, i, 0))


def _full_spec(shape):
    return pl.BlockSpec(shape, lambda i: tuple(0 for _ in shape))


def _vec_spec():
    return pl.BlockSpec((1, D), lambda i: (0, 0))


def kernel(x, x_org, edge_index, W_in, b_in, W_g0, b_g0, ln_g0, ln_b0,
           W_g1, b_g1, ln_g1, ln_b1, W_g2, b_g2, ln_g2, ln_b2,
           W_lin, b_lin, W_res):
    n = x.shape[0]
    e = edge_index.shape[1]
    src, dst = edge_index[0], edge_index[1]
    nch0 = -(-e // (NW * CH))
    nch = -(-nch0 // NB) * NB
    ep = nch * CH * NW
    src_p, dst_p = _pad_edges(src, dst, ep - e)
    src_p = src_p.reshape(NW * nch, CH)
    dst_p = dst_p.reshape(NW * nch, CH)

    grid = n // BR
    b_in2 = b_in.reshape(1, D)
    b_g02 = b_g0.reshape(1, D)
    b_g12 = b_g1.reshape(1, D)
    b_g22 = b_g2.reshape(1, D)
    g0 = ln_g0.reshape(1, D)
    be0 = ln_b0.reshape(1, D)
    g1 = ln_g1.reshape(1, D)
    be1 = ln_b1.reshape(1, D)
    g2 = ln_g2.reshape(1, D)
    be2 = ln_b2.reshape(1, D)
    b_lin2 = b_lin.reshape(1, D)

    zeros_blk = jnp.zeros((STRIPE, D), jnp.float32)
    ones_blk = jnp.ones((CH, D), jnp.float32)

    dg, = _make_sc_deg(nch)(dst_p, zeros_blk, ones_blk)

    xr, u1 = pl.pallas_call(
        _tc1_body,
        grid=(grid,),
        in_specs=[_row_spec(), _row_spec(), _full_spec((D, D)),
                  _full_spec((D, D)), _vec_spec(), _full_spec((D, D))],
        out_specs=[_row_spec(), _row_spec()],
        out_shape=[jax.ShapeDtypeStruct((n, D), jnp.float32),
                   jax.ShapeDtypeStruct((n, D), jnp.float32)],
    )(x, x_org, W_res, W_in, b_in2, W_g0)

    junk_blk = (N + (jnp.arange(NB * CH, dtype=jnp.int32)
                     % (NP - N))).reshape(NB, CH)
    sc_gs = _make_sc_scatter(nch)

    rp, = sc_gs(xr, src_p, dst_p, zeros_blk, junk_blk)

    res, t1, dinv = pl.pallas_call(
        _tc2_body,
        grid=(grid,),
        in_specs=[_part_spec(0), _part_spec(1),
                  _part_spec(0), _part_spec(1), _row_spec()],
        out_specs=[_row_spec(), _row_spec(), _row_spec()],
        out_shape=[jax.ShapeDtypeStruct((n, D), jnp.float32),
                   jax.ShapeDtypeStruct((n, D), jnp.float32),
                   jax.ShapeDtypeStruct((n, D), jnp.float32)],
    )(rp, rp, dg, dg, u1)

    def mid_layer(t, b2, g, be, wn):
        sp, = sc_gs(t, src_p, dst_p, zeros_blk, junk_blk)
        return pl.pallas_call(
            _tc_mid_body,
            grid=(grid,),
            in_specs=[_part_spec(0), _part_spec(1), _row_spec(), _row_spec(),
                      _vec_spec(), _vec_spec(), _vec_spec(),
                      _full_spec((D, D))],
            out_specs=[_row_spec()],
            out_shape=[jax.ShapeDtypeStruct((n, D), jnp.float32)],
        )(sp, sp, t, dinv, b2, g, be, wn)[0]

    t2 = mid_layer(t1, b_g02, g0, be0, W_g1)
    t3 = mid_layer(t2, b_g12, g1, be1, W_g2)

    sp, = sc_gs(t3, src_p, dst_p, zeros_blk, junk_blk)
    out = pl.pallas_call(
        _tc_fin_body,
        grid=(grid,),
        in_specs=[_part_spec(0), _part_spec(1), _row_spec(), _row_spec(),
                  _vec_spec(), _vec_spec(), _vec_spec(),
                  _full_spec((D, D)), _vec_spec()],
        out_specs=[_row_spec()],
        out_shape=[jax.ShapeDtypeStruct((n, D), jnp.float32)],
    )(sp, sp, t3, dinv, b_g22, g2, be2, W_lin, b_lin2)[0]

    return (out, res)


# speedup vs baseline: 1.1677x; 1.1565x over previous
import functools

import jax
import jax.numpy as jnp
from jax import lax
from jax.experimental import pallas as pl
from jax.experimental.pallas import tpu as pltpu
from jax.experimental.pallas import tpu_sc as plsc

N = 10000
D = 128
NC = 2
NS = 16
NW = NC * NS
CH = 64
NP = 10240
DUMMY_DST = N + 64
DEGW = 16


def _pad_edges(src, dst, e_pad):
    ar = jnp.arange(e_pad, dtype=jnp.int32)
    src_p = jnp.concatenate([src, ar % N])
    dst_p = jnp.concatenate([dst, N + (ar % (NP - N))])
    return src_p, dst_p


NB = 4
NQ = 4
STRIPE = NP // NS


def _sc_scatter_body(nch, table, src_hbm, dst_hbm, zeros_hbm, junk_hbm, pout,
                     src_v, dst_v, junk_v, r0_, r1_, r2_, r3_, acc_sh,
                     g0, g1, g2, g3, s0, s1, s2, s3):
    c = lax.axis_index("c")
    s = lax.axis_index("s")
    w = c * NS + s
    rows = [r0_, r1_, r2_, r3_]
    gsem = [g0, g1, g2, g3]
    ssem = [s0, s1, s2, s3]
    per_q = nch // NQ

    base_r = s * STRIPE
    pltpu.sync_copy(junk_hbm, junk_v)
    pltpu.sync_copy(zeros_hbm, acc_sh.at[pl.ds(base_r, STRIPE)])
    plsc.subcore_barrier()

    for b in range(NB):
        pltpu.async_copy(rows[b], acc_sh.at[junk_v.at[b]], ssem[b], add=True)

    def _quarter(qi, _):
        roff = w * nch + qi * per_q
        hs = pltpu.async_copy(src_hbm.at[pl.ds(roff, per_q)], src_v, g0)
        hd = pltpu.async_copy(dst_hbm.at[pl.ds(roff, per_q)], dst_v, g1)
        hs.wait()
        hd.wait()

        def _group(gi, _):
            hg = []
            for b in range(NB):
                j = gi * NB + b
                pltpu.make_async_copy(rows[b], acc_sh.at[dst_v.at[j]],
                                      ssem[b]).wait()
                hg.append(pltpu.async_copy(table.at[src_v.at[j]], rows[b],
                                           gsem[b]))
            for b in range(NB):
                j = gi * NB + b
                hg[b].wait()
                pltpu.async_copy(rows[b], acc_sh.at[dst_v.at[j]],
                                 ssem[b], add=True)
            return 0
        return lax.fori_loop(0, per_q // NB, _group, 0)
    lax.fori_loop(0, NQ, _quarter, 0)

    for b in range(NB):
        pltpu.make_async_copy(rows[b], acc_sh.at[junk_v.at[b]],
                              ssem[b]).wait()
    plsc.subcore_barrier()

    pltpu.sync_copy(acc_sh.at[pl.ds(base_r, STRIPE)],
                    pout.at[c, pl.ds(base_r, STRIPE)])


def _make_sc_scatter(nch):
    mesh = plsc.VectorSubcoreMesh(core_axis_name="c", subcore_axis_name="s")
    return pl.kernel(
        functools.partial(_sc_scatter_body, nch),
        out_type=[jax.ShapeDtypeStruct((NC, NP, D), jnp.float32)],
        mesh=mesh,
        scratch_types=[pltpu.VMEM((nch // NQ, CH), jnp.int32),
                       pltpu.VMEM((nch // NQ, CH), jnp.int32),
                       pltpu.VMEM((NB, CH), jnp.int32)]
                      + [pltpu.VMEM((CH, D), jnp.float32)] * NB
                      + [pltpu.VMEM_SHARED((NP, D), jnp.float32)]
                      + [pltpu.SemaphoreType.DMA] * (2 * NB))


def _sc_deg_body(nch, dst_hbm, zeros_hbm, ones_hbm, dout,
                 dst_v, ones_v, acc_sh, s0, s1, s2, s3):
    c = lax.axis_index("c")
    s = lax.axis_index("s")
    w = c * NS + s
    ssem = [s0, s1, s2, s3]
    per_q = nch // NQ

    base_r = s * STRIPE
    pltpu.sync_copy(zeros_hbm, acc_sh.at[pl.ds(base_r, STRIPE)])
    pltpu.sync_copy(ones_hbm, ones_v)
    plsc.subcore_barrier()

    def _quarter(qi, _):
        roff = w * nch + qi * per_q
        pltpu.sync_copy(dst_hbm.at[pl.ds(roff, per_q)], dst_v)

        def _group(gi, _):
            hsc = []
            for b in range(NB):
                j = gi * NB + b
                hsc.append(pltpu.async_copy(ones_v, acc_sh.at[dst_v.at[j]],
                                            ssem[b], add=True))
            for b in range(NB):
                hsc[b].wait()
            return 0
        return lax.fori_loop(0, per_q // NB, _group, 0)
    lax.fori_loop(0, NQ, _quarter, 0)
    plsc.subcore_barrier()

    pltpu.sync_copy(acc_sh.at[pl.ds(base_r, STRIPE)],
                    dout.at[c, pl.ds(base_r, STRIPE)])


def _make_sc_deg(nch):
    mesh = plsc.VectorSubcoreMesh(core_axis_name="c", subcore_axis_name="s")
    return pl.kernel(
        functools.partial(_sc_deg_body, nch),
        out_type=[jax.ShapeDtypeStruct((NC, NP, D), jnp.float32)],
        mesh=mesh,
        scratch_types=[pltpu.VMEM((nch // NQ, CH), jnp.int32),
                       pltpu.VMEM((CH, D), jnp.float32),
                       pltpu.VMEM_SHARED((NP, D), jnp.float32)]
                      + [pltpu.SemaphoreType.DMA] * NB)


BR = 400


def _tc1_body(x, xorg, wres, win, bin_, wg0, xr, u1):
    xr[...] = jnp.dot(xorg[...], wres[...], preferred_element_type=jnp.float32)
    h0 = jnp.maximum(
        jnp.dot(x[...], win[...], preferred_element_type=jnp.float32)
        + bin_[...], 0.0)
    u1[...] = jnp.dot(h0, wg0[...], preferred_element_type=jnp.float32)


def _tc2_body(rp0, rp1, dg0, dg1, u1, res, t1, dinv):
    res[...] = rp0[0] + rp1[0]
    deg = dg0[0][:, 0:1] + dg1[0][:, 0:1] + 1.0
    dv = lax.rsqrt(deg)
    dinv[...] = jnp.broadcast_to(dv, (BR, D))
    t1[...] = dv * u1[...]


def _layer_math(sp0, sp1, t, dinv, b, g, be):
    a = dinv[...] * (sp0[0] + sp1[0] + t[...]) + b[...]
    mu = jnp.mean(a, axis=-1, keepdims=True)
    var = jnp.mean((a - mu) ** 2, axis=-1, keepdims=True)
    xhat = (a - mu) * lax.rsqrt(var + 1e-5) * g[...] + be[...]
    return jnp.maximum(xhat, 0.0)


def _tc_mid_body(sp0, sp1, t, dinv, b, g, be, wn, tn):
    h = _layer_math(sp0, sp1, t, dinv, b, g, be)
    tn[...] = dinv[...] * jnp.dot(h, wn[...],
                                  preferred_element_type=jnp.float32)


def _tc_fin_body(sp0, sp1, t, dinv, b, g, be, wlin, blin, out):
    h = _layer_math(sp0, sp1, t, dinv, b, g, be)
    out[...] = (jnp.dot(h, wlin[...], preferred_element_type=jnp.float32)
                + blin[...])


def _row_spec():
    return pl.BlockSpec((BR, D), lambda i: (i, 0))


def _part_spec(core):
    return pl.BlockSpec((1, BR, D), lambda i, c=core: (c, i, 0))


def _full_spec(shape):
    return pl.BlockSpec(shape, lambda i: tuple(0 for _ in shape))


def _vec_spec():
    return pl.BlockSpec((1, D), lambda i: (0, 0))


def kernel(x, x_org, edge_index, W_in, b_in, W_g0, b_g0, ln_g0, ln_b0,
           W_g1, b_g1, ln_g1, ln_b1, W_g2, b_g2, ln_g2, ln_b2,
           W_lin, b_lin, W_res):
    n = x.shape[0]
    e = edge_index.shape[1]
    src, dst = edge_index[0], edge_index[1]
    nch0 = -(-e // (NW * CH))
    nch = -(-nch0 // NB) * NB
    ep = nch * CH * NW
    src_p, dst_p = _pad_edges(src, dst, ep - e)
    src_p = src_p.reshape(NW * nch, CH)
    dst_p = dst_p.reshape(NW * nch, CH)

    grid = n // BR
    b_in2 = b_in.reshape(1, D)
    b_g02 = b_g0.reshape(1, D)
    b_g12 = b_g1.reshape(1, D)
    b_g22 = b_g2.reshape(1, D)
    g0 = ln_g0.reshape(1, D)
    be0 = ln_b0.reshape(1, D)
    g1 = ln_g1.reshape(1, D)
    be1 = ln_b1.reshape(1, D)
    g2 = ln_g2.reshape(1, D)
    be2 = ln_b2.reshape(1, D)
    b_lin2 = b_lin.reshape(1, D)

    zeros_blk = jnp.zeros((STRIPE, D), jnp.float32)
    ones_blk = jnp.ones((CH, D), jnp.float32)

    dg, = _make_sc_deg(nch)(dst_p, zeros_blk, ones_blk)

    xr, u1 = pl.pallas_call(
        _tc1_body,
        grid=(grid,),
        in_specs=[_row_spec(), _row_spec(), _full_spec((D, D)),
                  _full_spec((D, D)), _vec_spec(), _full_spec((D, D))],
        out_specs=[_row_spec(), _row_spec()],
        out_shape=[jax.ShapeDtypeStruct((n, D), jnp.float32),
                   jax.ShapeDtypeStruct((n, D), jnp.float32)],
    )(x, x_org, W_res, W_in, b_in2, W_g0)

    junk_blk = (N + (jnp.arange(NB * CH, dtype=jnp.int32)
                     % (NP - N))).reshape(NB, CH)
    sc_gs = _make_sc_scatter(nch)

    rp, = sc_gs(xr, src_p, dst_p, zeros_blk, junk_blk)

    res, t1, dinv = pl.pallas_call(
        _tc2_body,
        grid=(grid,),
        in_specs=[_part_spec(0), _part_spec(1),
                  _part_spec(0), _part_spec(1), _row_spec()],
        out_specs=[_row_spec(), _row_spec(), _row_spec()],
        out_shape=[jax.ShapeDtypeStruct((n, D), jnp.float32),
                   jax.ShapeDtypeStruct((n, D), jnp.float32),
                   jax.ShapeDtypeStruct((n, D), jnp.float32)],
    )(rp, rp, dg, dg, u1)

    def mid_layer(t, b2, g, be, wn):
        sp, = sc_gs(t, src_p, dst_p, zeros_blk, junk_blk)
        return pl.pallas_call(
            _tc_mid_body,
            grid=(grid,),
            in_specs=[_part_spec(0), _part_spec(1), _row_spec(), _row_spec(),
                      _vec_spec(), _vec_spec(), _vec_spec(),
                      _full_spec((D, D))],
            out_specs=[_row_spec()],
            out_shape=[jax.ShapeDtypeStruct((n, D), jnp.float32)],
        )(sp, sp, t, dinv, b2, g, be, wn)[0]

    t2 = mid_layer(t1, b_g02, g0, be0, W_g1)
    t3 = mid_layer(t2, b_g12, g1, be1, W_g2)

    sp, = sc_gs(t3, src_p, dst_p, zeros_blk, junk_blk)
    out = pl.pallas_call(
        _tc_fin_body,
        grid=(grid,),
        in_specs=[_part_spec(0), _part_spec(1), _row_spec(), _row_spec(),
                  _vec_spec(), _vec_spec(), _vec_spec(),
                  _full_spec((D, D)), _vec_spec()],
        out_specs=[_row_spec()],
        out_shape=[jax.ShapeDtypeStruct((n, D), jnp.float32)],
    )(sp, sp, t3, dinv, b_g22, g2, be2, W_lin, b_lin2)[0]

    return (out, res)
